# baseline (speedup 1.0000x reference)
import functools
import math

import jax
import jax.numpy as jnp
import numpy as np
from jax.experimental import pallas as pl
from jax.experimental.pallas import tpu as pltpu

B = 8; L = 512; CIN = 8; PL_ = 96; D = 1024; E = 8; K = 2; HID = 2048
PATCH = 16; STRIDE = 8; PAD = 8
NPAT = 64
NF = D * NPAT
N = B * CIN * NPAT
C = int(N * 1.25 * K / E)
BC = 256


def _pos_embed():
    pos = np.arange(NPAT, dtype=np.float32)[:, None]
    div = np.exp(np.arange(0, D, 2, dtype=np.float32) * -(math.log(10000.0) / D))
    pe = np.zeros((NPAT, D), dtype=np.float32)
    pe[:, 0::2] = np.sin(pos * div)
    pe[:, 1::2] = np.cos(pos * div)
    return jnp.asarray(pe)


def _ffn_body(x_ref, w1_ref, b1_ref, w2_ref, b2_ref, o_ref):
    x = x_ref[0].astype(jnp.bfloat16)
    w1 = w1_ref[0].astype(jnp.bfloat16)
    h = jnp.dot(x, w1, preferred_element_type=jnp.float32) + b1_ref[0]
    h = jax.nn.gelu(h).astype(jnp.bfloat16)
    w2 = w2_ref[0].astype(jnp.bfloat16)
    o_ref[0] = jnp.dot(h, w2, preferred_element_type=jnp.float32) + b2_ref[0]


def _expert_ffn(buf, W1, b1, W2, b2):
    return pl.pallas_call(
        _ffn_body,
        grid=(E, C // BC),
        in_specs=[
            pl.BlockSpec((1, BC, D), lambda e, i: (e, i, 0)),
            pl.BlockSpec((1, D, HID), lambda e, i: (e, 0, 0)),
            pl.BlockSpec((1, 1, HID), lambda e, i: (e, 0, 0)),
            pl.BlockSpec((1, HID, D), lambda e, i: (e, 0, 0)),
            pl.BlockSpec((1, 1, D), lambda e, i: (e, 0, 0)),
        ],
        out_specs=pl.BlockSpec((1, BC, D), lambda e, i: (e, i, 0)),
        out_shape=jax.ShapeDtypeStruct((E, C, D), jnp.float32),
    )(buf, W1, b1.reshape(E, 1, HID), W2, b2.reshape(E, 1, D))


def kernel(x_enc, x_mark_enc, x_dec, x_mark_dec, W_patch, W_r, W1, b1, W2, b2, W_head, b_head):
    means = jnp.mean(x_enc, axis=1, keepdims=True)
    xe = x_enc - means
    stdev = jnp.sqrt(jnp.var(xe, axis=1, keepdims=True) + 1e-5)
    xe = xe / stdev
    x = jnp.transpose(xe, (0, 2, 1))
    x = jnp.pad(x, ((0, 0), (0, 0), (0, PAD)), mode='edge')
    idx = np.arange(NPAT)[:, None] * STRIDE + np.arange(PATCH)[None, :]
    patches = x[:, :, idx]
    tok = patches.reshape(B * CIN, NPAT, PATCH) @ W_patch + _pos_embed()[None]
    flat = tok.reshape(-1, D)

    logits = flat @ W_r
    probs = jax.nn.softmax(logits, axis=-1)
    gate_vals, expert_idx = jax.lax.top_k(probs, K)
    gate_vals = gate_vals / (jnp.sum(gate_vals, axis=-1, keepdims=True) + 1e-9)
    flat_e = expert_idx.reshape(-1)
    oh = jax.nn.one_hot(flat_e, E, dtype=jnp.float32)
    pos = jnp.sum((jnp.cumsum(oh, axis=0) - 1.0) * oh, axis=-1).astype(jnp.int32)
    keep = (pos < C).astype(flat.dtype)
    pos_c = jnp.minimum(pos, C - 1)
    x_rep = jnp.repeat(flat, K, axis=0)
    buf = jnp.zeros((E, C, D), flat.dtype).at[flat_e, pos_c].add(x_rep * keep[:, None])

    yb = _expert_ffn(buf, W1, b1, W2, b2)

    y_pair = yb[flat_e, pos_c] * (keep * gate_vals.reshape(-1))[:, None]
    y = jnp.sum(y_pair.reshape(N, K, D), axis=1)
    me = jnp.mean(probs, axis=0)
    ce = jnp.mean(jnp.sum(oh.reshape(N, K, E), axis=1), axis=0)
    balance = 0.01 * E * jnp.sum(me * ce)
    zloss = 0.001 * jnp.mean(jax.nn.logsumexp(logits, axis=-1) ** 2)
    aux = balance + zloss

    enc = y.reshape(B, CIN, NPAT, D)
    enc = jnp.transpose(enc, (0, 1, 3, 2))
    flat2 = enc.reshape(B, CIN, D * NPAT)
    dec = flat2 @ W_head + b_head
    dec = jnp.transpose(dec, (0, 2, 1))
    dec = dec * stdev[:, 0, :][:, None, :] + means[:, 0, :][:, None, :]
    return dec, aux
